# Initial kernel scaffold; baseline (speedup 1.0000x reference)
import jax
import jax.numpy as jnp
from jax.experimental import pallas as pl


def kernel(x, edge_index, emb, Wk1, bk1, Wq1, bq1, Wv1, bv1, Ws1, bo1, gamma1, beta1, Wk2, bk2, Wq2, bq2, Wv2, bv2, Ws2, bo2, gamma2, beta2, fcW, fcb):
    raise NotImplementedError("write your pallas kernel here")


# speedup vs baseline: 3.2678x; 3.2678x over previous
import functools
import jax
import jax.numpy as jnp
from jax import lax
from jax.experimental import pallas as pl
from jax.experimental.pallas import tpu as pltpu
from jax.experimental.pallas import tpu_sc as plsc

NC = 2
NS = 16
NW = NC * NS


def _emb_gather(emb, xpad, n_pad, d):
  rows_w = n_pad // NW
  ch = 80
  nch = rows_w // ch
  mesh = plsc.VectorSubcoreMesh(core_axis_name="c", subcore_axis_name="s")

  @functools.partial(
      pl.kernel,
      out_type=jax.ShapeDtypeStruct((n_pad, d), jnp.float32),
      mesh=mesh,
      scratch_types=[
          pltpu.VMEM((ch,), jnp.int32),
          pltpu.VMEM((ch, d), jnp.float32),
          pltpu.SemaphoreType.DMA,
      ],
  )
  def gather_k(emb_hbm, x_hbm, out_hbm, idx_v, rows_v, sem):
    wid = lax.axis_index("s") * NC + lax.axis_index("c")
    base = wid * rows_w

    def body(j, carry):
      off = base + j * ch
      pltpu.sync_copy(x_hbm.at[pl.ds(off, ch)], idx_v)
      pltpu.async_copy(emb_hbm.at[idx_v], rows_v, sem).wait()
      pltpu.sync_copy(rows_v, out_hbm.at[pl.ds(off, ch)])
      return carry

    lax.fori_loop(0, nch, body, 0)

  return gather_k(emb, xpad)


def _edge_pass(K, Q, V, srcp, dstg, dsts, n_pad, d, e_pad):
  ew = e_pad // NW
  c = 128
  nch = ew // c
  rows_acc = n_pad // NS
  mesh = plsc.VectorSubcoreMesh(core_axis_name="c", subcore_axis_name="s")
  zeros = jnp.zeros((rows_acc, d), jnp.float32)

  @functools.partial(
      pl.kernel,
      out_type=jax.ShapeDtypeStruct((NC, n_pad, d), jnp.float32),
      mesh=mesh,
      scratch_types=[
          pltpu.VMEM((c,), jnp.int32),
          pltpu.VMEM((c,), jnp.int32),
          pltpu.VMEM((c,), jnp.int32),
          pltpu.VMEM((c, d), jnp.float32),
          pltpu.VMEM((c, d), jnp.float32),
          pltpu.VMEM((c, d), jnp.float32),
          pltpu.VMEM_SHARED((n_pad, d), jnp.float32),
          pltpu.SemaphoreType.DMA,
          pltpu.SemaphoreType.DMA,
          pltpu.SemaphoreType.DMA,
      ],
  )
  def edge_k(k_hbm, q_hbm, v_hbm, src_hbm, dstg_hbm, dsts_hbm, z_hbm, out_hbm,
             sidx, didx, dsdx, kv, qv, vv, acc, s1, s2, s3):
    cid = lax.axis_index("c")
    sid = lax.axis_index("s")
    wid = sid * NC + cid

    pltpu.sync_copy(z_hbm, acc.at[pl.ds(sid * rows_acc, rows_acc)])
    plsc.subcore_barrier()

    def chunk(j, carry):
      base = wid * ew + j * c
      pltpu.sync_copy(src_hbm.at[pl.ds(base, c)], sidx)
      pltpu.sync_copy(dstg_hbm.at[pl.ds(base, c)], didx)
      pltpu.sync_copy(dsts_hbm.at[pl.ds(base, c)], dsdx)
      ck = pltpu.async_copy(k_hbm.at[didx], kv, s1)
      cq = pltpu.async_copy(q_hbm.at[sidx], qv, s2)
      cv = pltpu.async_copy(v_hbm.at[sidx], vv, s3)
      ck.wait()
      cq.wait()
      cv.wait()

      def row(r, rcarry):
        for cb in range(d // 16):
          sl = pl.ds(cb * 16, 16)
          g = kv[r, sl] + qv[r, sl]
          sig = 1.0 / (1.0 + jnp.exp(-g))
          vv[r, sl] = sig * vv[r, sl]
        return rcarry

      lax.fori_loop(0, c, row, 0)
      pltpu.sync_copy(vv, acc.at[dsdx], add=True)
      return carry

    lax.fori_loop(0, nch, chunk, 0)
    plsc.subcore_barrier()
    pltpu.sync_copy(acc.at[pl.ds(sid * rows_acc, rows_acc)],
                    out_hbm.at[cid, pl.ds(sid * rows_acc, rows_acc)])

  return edge_k(K, Q, V, srcp, dstg, dsts, zeros)


def _projections(h, wkT, bk, wqT, bq, wvT, bv, wsT, bo):
  n, d = h.shape
  br = 2000
  grid = n // br

  def body(h_ref, wk_r, bk_r, wq_r, bq_r, wv_r, bv_r, ws_r, bo_r,
           k_o, q_o, v_o, s_o):
    hh = h_ref[...]
    k_o[...] = jnp.dot(hh, wk_r[...], preferred_element_type=jnp.float32) + bk_r[...]
    q_o[...] = jnp.dot(hh, wq_r[...], preferred_element_type=jnp.float32) + bq_r[...]
    v_o[...] = jnp.dot(hh, wv_r[...], preferred_element_type=jnp.float32) + bv_r[...]
    s_o[...] = jnp.dot(hh, ws_r[...], preferred_element_type=jnp.float32) + bo_r[...]

  row_spec = pl.BlockSpec((br, d), lambda i: (i, 0))
  w_spec = pl.BlockSpec((d, d), lambda i: (0, 0))
  b_spec = pl.BlockSpec((1, d), lambda i: (0, 0))
  out = jax.ShapeDtypeStruct((n, d), jnp.float32)
  return pl.pallas_call(
      body,
      grid=(grid,),
      in_specs=[row_spec, w_spec, b_spec, w_spec, b_spec, w_spec, b_spec,
                w_spec, b_spec],
      out_specs=[row_spec, row_spec, row_spec, row_spec],
      out_shape=[out, out, out, out],
  )(h, wkT, bk.reshape(1, d), wqT, bq.reshape(1, d), wvT, bv.reshape(1, d),
    wsT, bo.reshape(1, d))


def kernel(x, edge_index, emb, Wk1, bk1, Wq1, bq1, Wv1, bv1, Ws1, bo1,
           gamma1, beta1, Wk2, bk2, Wq2, bq2, Wv2, bv2, Ws2, bo2,
           gamma2, beta2, fcW, fcb):
  n, d = emb.shape
  e = edge_index.shape[1]
  n_pad = 10240
  n_acc = 10112
  e_pad = 327680

  x = x.astype(jnp.int32)
  src = edge_index[0].astype(jnp.int32)
  dst = edge_index[1].astype(jnp.int32)
  xpad = jnp.concatenate([x, jnp.zeros((n_pad - n,), jnp.int32)])
  srcp = jnp.concatenate([src, jnp.zeros((e_pad - e,), jnp.int32)])
  dstg = jnp.concatenate([dst, jnp.zeros((e_pad - e,), jnp.int32)])
  dsts = jnp.concatenate([dst, jnp.full((e_pad - e,), n, jnp.int32)])

  h = _emb_gather(emb, xpad, n_pad, d)[:n]

  k1, q1, v1, s1 = _projections(h, Wk1.T, bk1, Wq1.T, bq1, Wv1.T, bv1,
                                Ws1.T, bo1)
  p1 = _edge_pass(k1, q1, v1, srcp, dstg, dsts, n_acc, d, e_pad)
  h1 = _combine_bn_relu_affine(p1, s1, gamma1, beta1)

  k2, q2, v2, s2 = _projections(h1, Wk2.T, bk2, Wq2.T, bq2, Wv2.T, bv2,
                                Ws2.T, bo2)
  p2 = _edge_pass(k2, q2, v2, srcp, dstg, dsts, n_acc, d, e_pad)
  return _final_affine(p2, s2, gamma2, beta2, fcW.T, fcb)


def _combine_bn_relu_affine(p, s, gamma, beta):
  n, d = s.shape

  def body(p_ref, s_ref, g_ref, b_ref, o_ref):
    pre = p_ref[0, :n, :] + p_ref[1, :n, :] + s_ref[...]
    mu = jnp.sum(pre, axis=0, keepdims=True) * (1.0 / n)
    cen = pre - mu
    var = jnp.sum(cen * cen, axis=0, keepdims=True) * (1.0 / n)
    o_ref[...] = jnp.maximum(
        cen * lax.rsqrt(var + 1e-5) * g_ref[...] + b_ref[...], 0.0)

  return pl.pallas_call(
      body,
      out_shape=jax.ShapeDtypeStruct((n, d), jnp.float32),
  )(p, s, gamma.reshape(1, d), beta.reshape(1, d))


def _final_affine(p, s, gamma, beta, fcT, fcb):
  n, d = s.shape

  def body(p_ref, s_ref, g_ref, b_ref, w_ref, fb_ref, o_ref):
    pre = p_ref[0, :n, :] + p_ref[1, :n, :] + s_ref[...]
    mu = jnp.sum(pre, axis=0, keepdims=True) * (1.0 / n)
    cen = pre - mu
    var = jnp.sum(cen * cen, axis=0, keepdims=True) * (1.0 / n)
    h = jnp.maximum(
        cen * lax.rsqrt(var + 1e-5) * g_ref[...] + b_ref[...], 0.0)
    o_ref[...] = jnp.dot(h, w_ref[...], preferred_element_type=jnp.float32) + fb_ref[...]

  return pl.pallas_call(
      body,
      out_shape=jax.ShapeDtypeStruct((n, d), jnp.float32),
  )(p, s, gamma.reshape(1, d), beta.reshape(1, d), fcT, fcb.reshape(1, d))


# speedup vs baseline: 4.5443x; 1.3906x over previous
import functools
import jax
import jax.numpy as jnp
from jax import lax
from jax.experimental import pallas as pl
from jax.experimental.pallas import tpu as pltpu
from jax.experimental.pallas import tpu_sc as plsc

NC = 2
NS = 16
NW = NC * NS


def _emb_gather(emb, xpad, n_pad, d):
  rows_w = n_pad // NW
  ch = 80
  nch = rows_w // ch
  mesh = plsc.VectorSubcoreMesh(core_axis_name="c", subcore_axis_name="s")

  @functools.partial(
      pl.kernel,
      out_type=jax.ShapeDtypeStruct((n_pad, d), jnp.float32),
      mesh=mesh,
      scratch_types=[
          pltpu.VMEM((ch,), jnp.int32),
          pltpu.VMEM((ch, d), jnp.float32),
          pltpu.SemaphoreType.DMA,
      ],
  )
  def gather_k(emb_hbm, x_hbm, out_hbm, idx_v, rows_v, sem):
    wid = lax.axis_index("s") * NC + lax.axis_index("c")
    base = wid * rows_w

    def body(j, carry):
      off = base + j * ch
      pltpu.sync_copy(x_hbm.at[pl.ds(off, ch)], idx_v)
      pltpu.async_copy(emb_hbm.at[idx_v], rows_v, sem).wait()
      pltpu.sync_copy(rows_v, out_hbm.at[pl.ds(off, ch)])
      return carry

    lax.fori_loop(0, nch, body, 0)

  return gather_k(emb, xpad)


def _edge_pass(K, Q, V, srcp, dstg, dsts, n_pad, d, e_pad):
  ew = e_pad // NW
  c = 64
  nch = ew // c
  rows_acc = n_pad // NS
  mesh = plsc.VectorSubcoreMesh(core_axis_name="c", subcore_axis_name="s")
  zeros = jnp.zeros((rows_acc, d), jnp.float32)

  @functools.partial(
      pl.kernel,
      out_type=jax.ShapeDtypeStruct((NC, n_pad, d), jnp.float32),
      mesh=mesh,
      scratch_types=[
          pltpu.VMEM((2, c), jnp.int32),
          pltpu.VMEM((2, c), jnp.int32),
          pltpu.VMEM((2, c), jnp.int32),
          pltpu.VMEM((2, c, d), jnp.float32),
          pltpu.VMEM((2, c, d), jnp.float32),
          pltpu.VMEM((2, c, d), jnp.float32),
          pltpu.VMEM_SHARED((n_pad, d), jnp.float32),
          pltpu.SemaphoreType.DMA,
          pltpu.SemaphoreType.DMA,
          pltpu.SemaphoreType.DMA,
          pltpu.SemaphoreType.DMA,
          pltpu.SemaphoreType.DMA,
          pltpu.SemaphoreType.DMA,
      ],
  )
  def edge_k(k_hbm, q_hbm, v_hbm, src_hbm, dstg_hbm, dsts_hbm, z_hbm, out_hbm,
             sidx, didx, dsdx, kv, qv, vv, acc,
             sk0, sk1, sq0, sq1, sv0, sv1):
    cid = lax.axis_index("c")
    sid = lax.axis_index("s")
    wid = sid * NC + cid
    base0 = wid * ew
    sks = (sk0, sk1)
    sqs = (sq0, sq1)
    svs = (sv0, sv1)

    pltpu.sync_copy(z_hbm, acc.at[pl.ds(sid * rows_acc, rows_acc)])
    plsc.subcore_barrier()

    def fire(j, b):
      base = base0 + j * c
      pltpu.sync_copy(src_hbm.at[pl.ds(base, c)], sidx.at[b])
      pltpu.sync_copy(dstg_hbm.at[pl.ds(base, c)], didx.at[b])
      pltpu.sync_copy(dsts_hbm.at[pl.ds(base, c)], dsdx.at[b])
      pltpu.async_copy(k_hbm.at[didx.at[b]], kv.at[b], sks[b])
      pltpu.async_copy(q_hbm.at[sidx.at[b]], qv.at[b], sqs[b])
      pltpu.async_copy(v_hbm.at[sidx.at[b]], vv.at[b], svs[b])

    for b in range(2):
      fire(b, b)

    def pair(jo, carry):
      for b in range(2):
        j = jo * 2 + b
        pltpu.make_async_copy(k_hbm.at[didx.at[b]], kv.at[b], sks[b]).wait()
        pltpu.make_async_copy(q_hbm.at[sidx.at[b]], qv.at[b], sqs[b]).wait()
        pltpu.make_async_copy(v_hbm.at[sidx.at[b]], vv.at[b], svs[b]).wait()

        def row(r, rc):
          for cb in range(d // 16):
            sl = pl.ds(cb * 16, 16)
            g = kv[b, r, sl] + qv[b, r, sl]
            sig = 1.0 / (1.0 + jnp.exp(-g))
            vv[b, r, sl] = sig * vv[b, r, sl]
          return rc
        lax.fori_loop(0, c, row, 0)

        pltpu.sync_copy(vv.at[b], acc.at[dsdx.at[b]], add=True)

        @pl.when(j + 2 < nch)
        def _():
          fire(j + 2, b)
      return carry

    lax.fori_loop(0, nch // 2, pair, 0)

    plsc.subcore_barrier()
    pltpu.sync_copy(acc.at[pl.ds(sid * rows_acc, rows_acc)],
                    out_hbm.at[cid, pl.ds(sid * rows_acc, rows_acc)])

  return edge_k(K, Q, V, srcp, dstg, dsts, zeros)


def _projections(h, wkT, bk, wqT, bq, wvT, bv, wsT, bo):
  n, d = h.shape
  br = 2000
  grid = n // br

  def body(h_ref, wk_r, bk_r, wq_r, bq_r, wv_r, bv_r, ws_r, bo_r,
           k_o, q_o, v_o, s_o):
    hh = h_ref[...]
    k_o[...] = jnp.dot(hh, wk_r[...], preferred_element_type=jnp.float32) + bk_r[...]
    q_o[...] = jnp.dot(hh, wq_r[...], preferred_element_type=jnp.float32) + bq_r[...]
    v_o[...] = jnp.dot(hh, wv_r[...], preferred_element_type=jnp.float32) + bv_r[...]
    s_o[...] = jnp.dot(hh, ws_r[...], preferred_element_type=jnp.float32) + bo_r[...]

  row_spec = pl.BlockSpec((br, d), lambda i: (i, 0))
  w_spec = pl.BlockSpec((d, d), lambda i: (0, 0))
  b_spec = pl.BlockSpec((1, d), lambda i: (0, 0))
  out = jax.ShapeDtypeStruct((n, d), jnp.float32)
  return pl.pallas_call(
      body,
      grid=(grid,),
      in_specs=[row_spec, w_spec, b_spec, w_spec, b_spec, w_spec, b_spec,
                w_spec, b_spec],
      out_specs=[row_spec, row_spec, row_spec, row_spec],
      out_shape=[out, out, out, out],
  )(h, wkT, bk.reshape(1, d), wqT, bq.reshape(1, d), wvT, bv.reshape(1, d),
    wsT, bo.reshape(1, d))


def kernel(x, edge_index, emb, Wk1, bk1, Wq1, bq1, Wv1, bv1, Ws1, bo1,
           gamma1, beta1, Wk2, bk2, Wq2, bq2, Wv2, bv2, Ws2, bo2,
           gamma2, beta2, fcW, fcb):
  n, d = emb.shape
  e = edge_index.shape[1]
  n_pad = 10240
  n_acc = 10112
  e_pad = 327680

  x = x.astype(jnp.int32)
  src = edge_index[0].astype(jnp.int32)
  dst = edge_index[1].astype(jnp.int32)
  xpad = jnp.concatenate([x, jnp.zeros((n_pad - n,), jnp.int32)])
  srcp = jnp.concatenate([src, jnp.zeros((e_pad - e,), jnp.int32)])
  dstg = jnp.concatenate([dst, jnp.zeros((e_pad - e,), jnp.int32)])
  dsts = jnp.concatenate([dst, jnp.full((e_pad - e,), n, jnp.int32)])

  h = _emb_gather(emb, xpad, n_pad, d)[:n]

  k1, q1, v1, s1 = _projections(h, Wk1.T, bk1, Wq1.T, bq1, Wv1.T, bv1,
                                Ws1.T, bo1)
  p1 = _edge_pass(k1, q1, v1, srcp, dstg, dsts, n_acc, d, e_pad)
  h1 = _combine_bn_relu_affine(p1, s1, gamma1, beta1)

  k2, q2, v2, s2 = _projections(h1, Wk2.T, bk2, Wq2.T, bq2, Wv2.T, bv2,
                                Ws2.T, bo2)
  p2 = _edge_pass(k2, q2, v2, srcp, dstg, dsts, n_acc, d, e_pad)
  return _final_affine(p2, s2, gamma2, beta2, fcW.T, fcb)


def _combine_bn_relu_affine(p, s, gamma, beta):
  n, d = s.shape

  def body(p_ref, s_ref, g_ref, b_ref, o_ref):
    pre = p_ref[0, :n, :] + p_ref[1, :n, :] + s_ref[...]
    mu = jnp.sum(pre, axis=0, keepdims=True) * (1.0 / n)
    cen = pre - mu
    var = jnp.sum(cen * cen, axis=0, keepdims=True) * (1.0 / n)
    o_ref[...] = jnp.maximum(
        cen * lax.rsqrt(var + 1e-5) * g_ref[...] + b_ref[...], 0.0)

  return pl.pallas_call(
      body,
      out_shape=jax.ShapeDtypeStruct((n, d), jnp.float32),
  )(p, s, gamma.reshape(1, d), beta.reshape(1, d))


def _final_affine(p, s, gamma, beta, fcT, fcb):
  n, d = s.shape

  def body(p_ref, s_ref, g_ref, b_ref, w_ref, fb_ref, o_ref):
    pre = p_ref[0, :n, :] + p_ref[1, :n, :] + s_ref[...]
    mu = jnp.sum(pre, axis=0, keepdims=True) * (1.0 / n)
    cen = pre - mu
    var = jnp.sum(cen * cen, axis=0, keepdims=True) * (1.0 / n)
    h = jnp.maximum(
        cen * lax.rsqrt(var + 1e-5) * g_ref[...] + b_ref[...], 0.0)
    o_ref[...] = jnp.dot(h, w_ref[...], preferred_element_type=jnp.float32) + fb_ref[...]

  return pl.pallas_call(
      body,
      out_shape=jax.ShapeDtypeStruct((n, d), jnp.float32),
  )(p, s, gamma.reshape(1, d), beta.reshape(1, d), fcT, fcb.reshape(1, d))
